# baseline (speedup 1.0000x reference)
import functools

import jax
import jax.numpy as jnp
import numpy as np
from jax import lax
from jax.experimental import pallas as pl
from jax.experimental.pallas import tpu as pltpu
from jax.experimental.pallas import tpu_sc as plsc

N = 10000
NPAD = 10240
E = 160000
EP = 163840
NTILES = 32
TOTC = EP // 128
K0 = 40
K1 = (TOTC - 16 * K0) // 16
NCHD = TOTC // NTILES
ZR = NPAD // 16
NB = 512

T0, TIN, HID, KS = 10, 2, 16, 3
TP = T0 - KS + 1
OUTF = 5
TQ = TP - KS + 1
FILT = 32
F1 = TP * HID
F2 = 32
FZ = TQ * FILT


def _sc_mesh():
    return plsc.VectorSubcoreMesh(core_axis_name="c", subcore_axis_name="s")


@functools.lru_cache(maxsize=None)
def _make_sc_gather_scatter(feat):
    KM = max(K0, K1)

    @functools.partial(
        pl.kernel,
        out_type=jax.ShapeDtypeStruct((2, NPAD, feat), jnp.float32),
        mesh=_sc_mesh(),
        compiler_params=pltpu.CompilerParams(
            use_tc_tiling_on_sc=(feat % 128 == 0)),
        scratch_types=[
            pltpu.VMEM((KM, 128), jnp.int32),
            pltpu.VMEM((KM, 128), jnp.int32),
            pltpu.VMEM((128, feat), jnp.float32),
            pltpu.VMEM((128, feat), jnp.float32),
            pltpu.VMEM_SHARED((NPAD, feat), jnp.float32),
            pltpu.SemaphoreType.DMA,
            pltpu.SemaphoreType.DMA,
        ],
    )
    def k(u_hbm, row_hbm, col_hbm, zeros_hbm, out_hbm,
          row_v, col_v, buf0, buf1, acc, gs0, gs1):
        cid = lax.axis_index("c")
        sid = lax.axis_index("s")
        pltpu.sync_copy(zeros_hbm.at[pl.ds(sid * ZR, ZR)], acc.at[pl.ds(sid * ZR, ZR)])

        def run(nch, start):
            pltpu.sync_copy(row_hbm.at[pl.ds(start, nch)], row_v.at[pl.ds(0, nch)])
            pltpu.sync_copy(col_hbm.at[pl.ds(start, nch)], col_v.at[pl.ds(0, nch)])
            plsc.subcore_barrier()

            pltpu.async_copy(u_hbm.at[row_v.at[0]], buf0, gs0)

            def body(i, carry):
                j0 = 2 * i
                j1 = 2 * i + 1
                pltpu.make_async_copy(u_hbm.at[row_v.at[j0]], buf0, gs0).wait()
                pltpu.async_copy(u_hbm.at[row_v.at[j1]], buf1, gs1)
                pltpu.sync_copy(buf0, acc.at[col_v.at[j0]], add=True)
                pltpu.make_async_copy(u_hbm.at[row_v.at[j1]], buf1, gs1).wait()

                @pl.when(i < nch // 2 - 1)
                def _():
                    pltpu.async_copy(u_hbm.at[row_v.at[j0 + 2]], buf0, gs0)

                pltpu.sync_copy(buf1, acc.at[col_v.at[j1]], add=True)
                return carry

            lax.fori_loop(0, nch // 2, body, 0)

        @pl.when(cid == 0)
        def _():
            run(K0, sid * K0)

        @pl.when(cid == 1)
        def _():
            run(K1, 16 * K0 + sid * K1)

        plsc.subcore_barrier()
        pltpu.sync_copy(acc.at[pl.ds(sid * ZR, ZR)],
                        out_hbm.at[cid, pl.ds(sid * ZR, ZR)])

    return k


@functools.lru_cache(maxsize=None)
def _make_sc_degree():
    @functools.partial(
        pl.kernel,
        out_type=jax.ShapeDtypeStruct((2, NPAD, 16), jnp.float32),
        mesh=_sc_mesh(),
        compiler_params=pltpu.CompilerParams(use_tc_tiling_on_sc=False),
        scratch_types=[
            pltpu.VMEM((NCHD, 128), jnp.int32),
            pltpu.VMEM((128, 16), jnp.float32),
            pltpu.VMEM_SHARED((NPAD, 16), jnp.float32),
            pltpu.SemaphoreType.DMA,
        ],
    )
    def k(row_hbm, ones_hbm, zeros_hbm, out_hbm, row_v, ones_v, acc, sem):
        cid = lax.axis_index("c")
        sid = lax.axis_index("s")
        w = sid * 2 + cid
        pltpu.sync_copy(zeros_hbm.at[pl.ds(sid * ZR, ZR)], acc.at[pl.ds(sid * ZR, ZR)])
        pltpu.sync_copy(ones_hbm, ones_v)
        pltpu.sync_copy(row_hbm.at[pl.ds(w * NCHD, NCHD)], row_v)
        plsc.subcore_barrier()

        def body(j, carry):
            pltpu.sync_copy(ones_v, acc.at[row_v.at[j]], add=True)
            return carry

        lax.fori_loop(0, NCHD, body, 0)
        plsc.subcore_barrier()
        pltpu.sync_copy(acc.at[pl.ds(sid * ZR, ZR)],
                        out_hbm.at[cid, pl.ds(sid * ZR, ZR)])

    return k


def _sc_degree(row3, ones16, zeros16):
    return _make_sc_degree()(row3, ones16, zeros16)


def _sc_g128(u, row3, col3, zeros):
    return _make_sc_gather_scatter(F1)(u, row3, col3, zeros)


def _sc_g32(u, row3, col3, zeros):
    return _make_sc_gather_scatter(F2)(u, row3, col3, zeros)


def _dis_from(degp):
    d = degp[0, :, 0:1] + degp[1, :, 0:1]
    return jnp.where(d > 0, lax.rsqrt(d), 0.0)


def _tc_a_body(xn, degp, w1p, b1p, w1q, b1q, w1r, b1r, u0_ref, t0_ref):
    x = xn[...]
    dis = _dis_from(degp[...])
    P = jnp.dot(x, w1p[...], preferred_element_type=jnp.float32) + b1p[...]
    Q = jnp.dot(x, w1q[...], preferred_element_type=jnp.float32) + b1q[...]
    Rm = jnp.dot(x, w1r[...], preferred_element_type=jnp.float32) + b1r[...]
    t0 = jax.nn.relu(P * jax.nn.sigmoid(Q) + Rm)
    t0_ref[...] = t0
    u0_ref[...] = dis * t0


def _tc_b_body(g1p, t0, degp, w0bd, w1bd, u1_ref, o01_ref):
    dis = _dis_from(degp[...])
    g1 = g1p[0] + g1p[1]
    tx1 = -dis * g1
    u1_ref[...] = dis * tx1
    o01_ref[...] = (
        jnp.dot(t0[...], w0bd[...], preferred_element_type=jnp.float32)
        + jnp.dot(tx1, w1bd[...], preferred_element_type=jnp.float32)
    )


def _tc_c_body(g2p, t0, o01, degp, bnp, w2bd, bch, w2p, b2p, w2q, b2q, w2r, b2r,
               h_ref, uh_ref):
    dis = _dis_from(degp[...])
    tx2 = -2.0 * dis * (g2p[0] + g2p[1]) - t0[...]
    gfull = jax.nn.relu(
        o01[...] + jnp.dot(tx2, w2bd[...], preferred_element_type=jnp.float32)
        + bch[...]
    )
    P2 = jnp.dot(gfull, w2p[...], preferred_element_type=jnp.float32) + b2p[...]
    Q2 = jnp.dot(gfull, w2q[...], preferred_element_type=jnp.float32) + b2q[...]
    R2 = jnp.dot(gfull, w2r[...], preferred_element_type=jnp.float32) + b2r[...]
    t2 = jax.nn.relu(P2 * jax.nn.sigmoid(Q2) + R2)
    mask = (lax.broadcasted_iota(jnp.int32, (1, F2), 1) < 30).astype(jnp.float32)
    mean = jnp.sum(t2, axis=1, keepdims=True) * (1.0 / 30.0)
    ctr = t2 - mean
    var = jnp.sum(ctr * ctr * mask, axis=1, keepdims=True) * (1.0 / 30.0)
    hh = ctr * lax.rsqrt(var + 1e-5)
    h = jax.nn.relu(hh * bnp[:, 0:1] + bnp[:, 1:2])
    h_ref[...] = h
    uh_ref[...] = dis * h


def _tc_d_body(h, ghp, degp, wz0, wz1, bz, wh0, wh1, bh, lwbd, blin, res_ref):
    dis = _dis_from(degp[...])
    ph = -dis * (ghp[0] + ghp[1])
    hv = h[...]
    Z = jax.nn.sigmoid(
        jnp.dot(hv, wz0[...], preferred_element_type=jnp.float32)
        + jnp.dot(ph, wz1[...], preferred_element_type=jnp.float32) + bz[...]
    )
    Ht = jnp.tanh(
        jnp.dot(hv, wh0[...], preferred_element_type=jnp.float32)
        + jnp.dot(ph, wh1[...], preferred_element_type=jnp.float32) + bh[...]
    )
    H = jax.nn.relu((1.0 - Z) * Ht)
    o = jnp.dot(H, lwbd[...], preferred_element_type=jnp.float32) + blin[...]
    mask6 = lax.broadcasted_iota(jnp.int32, (1, 8), 1) < TQ
    om = jnp.where(mask6, o, -1e30)
    mx = jnp.max(om, axis=1, keepdims=True)
    lse = jnp.log(jnp.sum(jnp.exp(om - mx), axis=1, keepdims=True)) + mx
    res_ref[...] = o - lse


def _node_spec(feat):
    return pl.BlockSpec((NB, feat), lambda i: (i, 0))


def _part_spec(feat):
    return pl.BlockSpec((2, NB, feat), lambda i: (0, i, 0))


def _full_spec(shape):
    nd = len(shape)
    return pl.BlockSpec(shape, lambda i: (0,) * nd)


def _tc_call(body, in_arrays, in_specs, out_feats):
    out_shape = [jax.ShapeDtypeStruct((NPAD, f), jnp.float32) for f in out_feats]
    out_specs = [_node_spec(f) for f in out_feats]
    res = pl.pallas_call(
        body,
        grid=(NPAD // NB,),
        in_specs=in_specs,
        out_specs=out_specs,
        out_shape=out_shape,
    )(*in_arrays)
    return res


def _big_conv_w(w, t_in, c_in, t_out, c_out, pad_to=None):
    rows, cols, oo, ii, kk = [], [], [], [], []
    for ti in range(t_out):
        for k in range(KS):
            for i in range(c_in):
                for o in range(c_out):
                    rows.append((ti + k) * c_in + i)
                    cols.append(ti * c_out + o)
                    oo.append(o)
                    ii.append(i)
                    kk.append(k)
    vals = w[np.array(oo), np.array(ii), 0, np.array(kk)]
    ncol = t_out * c_out if pad_to is None else pad_to
    big = jnp.zeros((t_in * c_in, ncol), jnp.float32)
    return big.at[np.array(rows), np.array(cols)].set(vals)


def _tile_bias(b, t_out, pad_to=None):
    v = jnp.tile(b, t_out)
    if pad_to is not None and pad_to > v.shape[0]:
        v = jnp.pad(v, (0, pad_to - v.shape[0]))
    return v.reshape(1, -1)


def _block_diag(wmat, t, pad_rows=None, pad_cols=None):
    bd = jnp.kron(jnp.eye(t, dtype=jnp.float32), wmat)
    pr = 0 if pad_rows is None else pad_rows - bd.shape[0]
    pc = 0 if pad_cols is None else pad_cols - bd.shape[1]
    if pr or pc:
        bd = jnp.pad(bd, ((0, pr), (0, pc)))
    return bd


def kernel(x, params, edge_index):
    p = params
    f32 = jnp.float32

    pad = jnp.full((EP - E,), N, dtype=jnp.int32)
    perm = jnp.argsort(edge_index[0])
    row3 = jnp.concatenate([edge_index[0][perm], pad]).reshape(TOTC, 128)
    col3 = jnp.concatenate([edge_index[1][perm], pad]).reshape(TOTC, 128)
    xn = jnp.transpose(x[0], (1, 0, 2)).reshape(N, T0 * TIN)
    xn = jnp.pad(xn, ((0, NPAD - N), (0, 0)))
    zeros128 = jnp.zeros((NPAD, F1), f32)
    zeros32 = jnp.zeros((NPAD, F2), f32)
    zeros16 = jnp.zeros((NPAD, 16), f32)
    ones16 = jnp.ones((128, 16), f32)
    bnp = jnp.zeros((NPAD, 16), f32)
    bnp = bnp.at[:N, 0].set(p["bn_w"]).at[:N, 1].set(p["bn_b"])

    w1p = _big_conv_w(p["tc1_w1"], T0, TIN, TP, HID)
    w1q = _big_conv_w(p["tc1_w2"], T0, TIN, TP, HID)
    w1r = _big_conv_w(p["tc1_w3"], T0, TIN, TP, HID)
    b1p = _tile_bias(p["tc1_b1"], TP)
    b1q = _tile_bias(p["tc1_b2"], TP)
    b1r = _tile_bias(p["tc1_b3"], TP)
    w0bd = _block_diag(p["cheb1_W"][0], TP)
    w1bd = _block_diag(p["cheb1_W"][1], TP)
    w2bd = _block_diag(p["cheb1_W"][2], TP)
    bch = _tile_bias(p["cheb1_b"], TP)
    w2p = _big_conv_w(p["tc2_w1"], TP, HID, TQ, OUTF, pad_to=F2)
    w2q = _big_conv_w(p["tc2_w2"], TP, HID, TQ, OUTF, pad_to=F2)
    w2r = _big_conv_w(p["tc2_w3"], TP, HID, TQ, OUTF, pad_to=F2)
    b2p = _tile_bias(p["tc2_b1"], TQ, pad_to=F2)
    b2q = _tile_bias(p["tc2_b2"], TQ, pad_to=F2)
    b2r = _tile_bias(p["tc2_b3"], TQ, pad_to=F2)
    wz0 = _block_diag(p["Wxz"][0], TQ, pad_rows=F2)
    wz1 = _block_diag(p["Wxz"][1], TQ, pad_rows=F2)
    bz = _tile_bias(p["bxz"] + p["bhz"], TQ)
    wh0 = _block_diag(p["Wxh"][0], TQ, pad_rows=F2)
    wh1 = _block_diag(p["Wxh"][1], TQ, pad_rows=F2)
    bh = _tile_bias(p["bxh"] + p["bhh"], TQ)
    lwbd = _block_diag(p["lin_W"], TQ, pad_cols=8)
    blin = jnp.broadcast_to(p["lin_b"], (8,)).reshape(1, 8)

    degp = _sc_degree(row3, ones16, zeros16)

    u0, t0 = _tc_call(
        _tc_a_body,
        [xn, degp, w1p, b1p, w1q, b1q, w1r, b1r],
        [_node_spec(T0 * TIN), _part_spec(16),
         _full_spec((T0 * TIN, F1)), _full_spec((1, F1)),
         _full_spec((T0 * TIN, F1)), _full_spec((1, F1)),
         _full_spec((T0 * TIN, F1)), _full_spec((1, F1))],
        [F1, F1],
    )

    g1p = _sc_g128(u0, row3, col3, zeros128)

    u1, o01 = _tc_call(
        _tc_b_body,
        [g1p, t0, degp, w0bd, w1bd],
        [_part_spec(F1), _node_spec(F1), _part_spec(16),
         _full_spec((F1, F1)), _full_spec((F1, F1))],
        [F1, F1],
    )

    g2p = _sc_g128(u1, row3, col3, zeros128)

    h, uh = _tc_call(
        _tc_c_body,
        [g2p, t0, o01, degp, bnp, w2bd, bch, w2p, b2p, w2q, b2q, w2r, b2r],
        [_part_spec(F1), _node_spec(F1), _node_spec(F1), _part_spec(16),
         _node_spec(16), _full_spec((F1, F1)), _full_spec((1, F1)),
         _full_spec((F1, F2)), _full_spec((1, F2)),
         _full_spec((F1, F2)), _full_spec((1, F2)),
         _full_spec((F1, F2)), _full_spec((1, F2))],
        [F2, F2],
    )

    ghp = _sc_g32(uh, row3, col3, zeros32)

    (res,) = _tc_call(
        _tc_d_body,
        [h, ghp, degp, wz0, wz1, bz, wh0, wh1, bh, lwbd, blin],
        [_node_spec(F2), _part_spec(F2), _part_spec(16),
         _full_spec((F2, FZ)), _full_spec((F2, FZ)), _full_spec((1, FZ)),
         _full_spec((F2, FZ)), _full_spec((F2, FZ)), _full_spec((1, FZ)),
         _full_spec((FZ, 8)), _full_spec((1, 8))],
        [8],
    )

    out = res[:N, :TQ]
    return jnp.transpose(out, (1, 0))[None, :, :, None]


# speedup vs baseline: 1.3880x; 1.3880x over previous
import functools

import jax
import jax.numpy as jnp
import numpy as np
from jax import lax
from jax.experimental import pallas as pl
from jax.experimental.pallas import tpu as pltpu
from jax.experimental.pallas import tpu_sc as plsc

N = 10000
NPAD = 10240
E = 160000
EP = 163840
NTILES = 32
NCHUNK = EP // (NTILES * 128)
ZR = NPAD // 16
NB = 512

T0, TIN, HID, KS = 10, 2, 16, 3
TP = T0 - KS + 1
OUTF = 5
TQ = TP - KS + 1
FILT = 32
F1 = TP * HID
F2 = 32
FZ = TQ * FILT


def _sc_mesh():
    return plsc.VectorSubcoreMesh(core_axis_name="c", subcore_axis_name="s")


@functools.lru_cache(maxsize=None)
def _make_sc_gather_scatter(feat):
    @functools.partial(
        pl.kernel,
        out_type=jax.ShapeDtypeStruct((2, NPAD, feat), jnp.float32),
        mesh=_sc_mesh(),
        compiler_params=pltpu.CompilerParams(
            use_tc_tiling_on_sc=(feat % 128 == 0)),
        scratch_types=[
            pltpu.VMEM((NCHUNK, 128), jnp.int32),
            pltpu.VMEM((NCHUNK, 128), jnp.int32),
            pltpu.VMEM((128, feat), jnp.float32),
            pltpu.VMEM((128, feat), jnp.float32),
            pltpu.VMEM_SHARED((NPAD, feat), jnp.float32),
            pltpu.SemaphoreType.DMA,
            pltpu.SemaphoreType.DMA,
        ],
    )
    def k(u_hbm, row_hbm, col_hbm, zeros_hbm, out_hbm,
          row_v, col_v, buf0, buf1, acc, gs0, gs1):
        cid = lax.axis_index("c")
        sid = lax.axis_index("s")
        w = sid * 2 + cid
        pltpu.sync_copy(zeros_hbm.at[pl.ds(sid * ZR, ZR)], acc.at[pl.ds(sid * ZR, ZR)])
        pltpu.sync_copy(row_hbm.at[w], row_v)
        pltpu.sync_copy(col_hbm.at[w], col_v)
        plsc.subcore_barrier()

        pltpu.async_copy(u_hbm.at[row_v.at[0]], buf0, gs0)

        def body(i, carry):
            j0 = 2 * i
            j1 = 2 * i + 1
            pltpu.make_async_copy(u_hbm.at[row_v.at[j0]], buf0, gs0).wait()
            pltpu.async_copy(u_hbm.at[row_v.at[j1]], buf1, gs1)
            pltpu.sync_copy(buf0, acc.at[col_v.at[j0]], add=True)
            pltpu.make_async_copy(u_hbm.at[row_v.at[j1]], buf1, gs1).wait()

            @pl.when(i < NCHUNK // 2 - 1)
            def _():
                pltpu.async_copy(u_hbm.at[row_v.at[j0 + 2]], buf0, gs0)

            pltpu.sync_copy(buf1, acc.at[col_v.at[j1]], add=True)
            return carry

        lax.fori_loop(0, NCHUNK // 2, body, 0)
        plsc.subcore_barrier()
        pltpu.sync_copy(acc.at[pl.ds(sid * ZR, ZR)],
                        out_hbm.at[cid, pl.ds(sid * ZR, ZR)])

    return k


@functools.lru_cache(maxsize=None)
def _make_sc_degree():
    @functools.partial(
        pl.kernel,
        out_type=jax.ShapeDtypeStruct((2, NPAD, 16), jnp.float32),
        mesh=_sc_mesh(),
        compiler_params=pltpu.CompilerParams(use_tc_tiling_on_sc=False),
        scratch_types=[
            pltpu.VMEM((NCHUNK, 128), jnp.int32),
            pltpu.VMEM((128, 16), jnp.float32),
            pltpu.VMEM_SHARED((NPAD, 16), jnp.float32),
            pltpu.SemaphoreType.DMA,
        ],
    )
    def k(row_hbm, ones_hbm, zeros_hbm, out_hbm, row_v, ones_v, acc, sem):
        cid = lax.axis_index("c")
        sid = lax.axis_index("s")
        w = sid * 2 + cid
        pltpu.sync_copy(zeros_hbm.at[pl.ds(sid * ZR, ZR)], acc.at[pl.ds(sid * ZR, ZR)])
        pltpu.sync_copy(ones_hbm, ones_v)
        pltpu.sync_copy(row_hbm.at[w], row_v)
        plsc.subcore_barrier()

        def body(j, carry):
            pltpu.sync_copy(ones_v, acc.at[row_v.at[j]], add=True)
            return carry

        lax.fori_loop(0, NCHUNK, body, 0)
        plsc.subcore_barrier()
        pltpu.sync_copy(acc.at[pl.ds(sid * ZR, ZR)],
                        out_hbm.at[cid, pl.ds(sid * ZR, ZR)])

    return k


def _sc_degree(row3, ones16, zeros16):
    return _make_sc_degree()(row3, ones16, zeros16)


def _sc_g128(u, row3, col3, zeros):
    return _make_sc_gather_scatter(F1)(u, row3, col3, zeros)


def _sc_g32(u, row3, col3, zeros):
    return _make_sc_gather_scatter(F2)(u, row3, col3, zeros)


def _dis_from(degp):
    d = degp[0, :, 0:1] + degp[1, :, 0:1]
    return jnp.where(d > 0, lax.rsqrt(d), 0.0)


def _tc_a_body(xn, degp, w1p, b1p, w1q, b1q, w1r, b1r, u0_ref, t0_ref):
    x = xn[...]
    dis = _dis_from(degp[...])
    P = jnp.dot(x, w1p[...], preferred_element_type=jnp.float32) + b1p[...]
    Q = jnp.dot(x, w1q[...], preferred_element_type=jnp.float32) + b1q[...]
    Rm = jnp.dot(x, w1r[...], preferred_element_type=jnp.float32) + b1r[...]
    t0 = jax.nn.relu(P * jax.nn.sigmoid(Q) + Rm)
    t0_ref[...] = t0
    u0_ref[...] = dis * t0


def _tc_b_body(g1p, t0, degp, w0bd, w1bd, u1_ref, o01_ref):
    dis = _dis_from(degp[...])
    g1 = g1p[0] + g1p[1]
    tx1 = -dis * g1
    u1_ref[...] = dis * tx1
    o01_ref[...] = (
        jnp.dot(t0[...], w0bd[...], preferred_element_type=jnp.float32)
        + jnp.dot(tx1, w1bd[...], preferred_element_type=jnp.float32)
    )


def _tc_c_body(g2p, t0, o01, degp, bnp, w2bd, bch, w2p, b2p, w2q, b2q, w2r, b2r,
               h_ref, uh_ref):
    dis = _dis_from(degp[...])
    tx2 = -2.0 * dis * (g2p[0] + g2p[1]) - t0[...]
    gfull = jax.nn.relu(
        o01[...] + jnp.dot(tx2, w2bd[...], preferred_element_type=jnp.float32)
        + bch[...]
    )
    P2 = jnp.dot(gfull, w2p[...], preferred_element_type=jnp.float32) + b2p[...]
    Q2 = jnp.dot(gfull, w2q[...], preferred_element_type=jnp.float32) + b2q[...]
    R2 = jnp.dot(gfull, w2r[...], preferred_element_type=jnp.float32) + b2r[...]
    t2 = jax.nn.relu(P2 * jax.nn.sigmoid(Q2) + R2)
    mask = (lax.broadcasted_iota(jnp.int32, (1, F2), 1) < 30).astype(jnp.float32)
    mean = jnp.sum(t2, axis=1, keepdims=True) * (1.0 / 30.0)
    ctr = t2 - mean
    var = jnp.sum(ctr * ctr * mask, axis=1, keepdims=True) * (1.0 / 30.0)
    hh = ctr * lax.rsqrt(var + 1e-5)
    h = jax.nn.relu(hh * bnp[:, 0:1] + bnp[:, 1:2])
    h_ref[...] = h
    uh_ref[...] = dis * h


def _tc_d_body(h, ghp, degp, wz0, wz1, bz, wh0, wh1, bh, lwbd, blin, res_ref):
    dis = _dis_from(degp[...])
    ph = -dis * (ghp[0] + ghp[1])
    hv = h[...]
    Z = jax.nn.sigmoid(
        jnp.dot(hv, wz0[...], preferred_element_type=jnp.float32)
        + jnp.dot(ph, wz1[...], preferred_element_type=jnp.float32) + bz[...]
    )
    Ht = jnp.tanh(
        jnp.dot(hv, wh0[...], preferred_element_type=jnp.float32)
        + jnp.dot(ph, wh1[...], preferred_element_type=jnp.float32) + bh[...]
    )
    H = jax.nn.relu((1.0 - Z) * Ht)
    o = jnp.dot(H, lwbd[...], preferred_element_type=jnp.float32) + blin[...]
    mask6 = lax.broadcasted_iota(jnp.int32, (1, 8), 1) < TQ
    om = jnp.where(mask6, o, -1e30)
    mx = jnp.max(om, axis=1, keepdims=True)
    lse = jnp.log(jnp.sum(jnp.exp(om - mx), axis=1, keepdims=True)) + mx
    res_ref[...] = o - lse


def _node_spec(feat):
    return pl.BlockSpec((NB, feat), lambda i: (i, 0))


def _part_spec(feat):
    return pl.BlockSpec((2, NB, feat), lambda i: (0, i, 0))


def _full_spec(shape):
    nd = len(shape)
    return pl.BlockSpec(shape, lambda i: (0,) * nd)


def _tc_call(body, in_arrays, in_specs, out_feats):
    out_shape = [jax.ShapeDtypeStruct((NPAD, f), jnp.float32) for f in out_feats]
    out_specs = [_node_spec(f) for f in out_feats]
    res = pl.pallas_call(
        body,
        grid=(NPAD // NB,),
        in_specs=in_specs,
        out_specs=out_specs,
        out_shape=out_shape,
    )(*in_arrays)
    return res


def _big_conv_w(w, t_in, c_in, t_out, c_out, pad_to=None):
    rows, cols, oo, ii, kk = [], [], [], [], []
    for ti in range(t_out):
        for k in range(KS):
            for i in range(c_in):
                for o in range(c_out):
                    rows.append((ti + k) * c_in + i)
                    cols.append(ti * c_out + o)
                    oo.append(o)
                    ii.append(i)
                    kk.append(k)
    vals = w[np.array(oo), np.array(ii), 0, np.array(kk)]
    ncol = t_out * c_out if pad_to is None else pad_to
    big = jnp.zeros((t_in * c_in, ncol), jnp.float32)
    return big.at[np.array(rows), np.array(cols)].set(vals)


def _tile_bias(b, t_out, pad_to=None):
    v = jnp.tile(b, t_out)
    if pad_to is not None and pad_to > v.shape[0]:
        v = jnp.pad(v, (0, pad_to - v.shape[0]))
    return v.reshape(1, -1)


def _block_diag(wmat, t, pad_rows=None, pad_cols=None):
    bd = jnp.kron(jnp.eye(t, dtype=jnp.float32), wmat)
    pr = 0 if pad_rows is None else pad_rows - bd.shape[0]
    pc = 0 if pad_cols is None else pad_cols - bd.shape[1]
    if pr or pc:
        bd = jnp.pad(bd, ((0, pr), (0, pc)))
    return bd


def kernel(x, params, edge_index):
    p = params
    f32 = jnp.float32

    pad = jnp.full((EP - E,), N, dtype=jnp.int32)
    row3 = jnp.concatenate([edge_index[0], pad]).reshape(NTILES, NCHUNK, 128)
    col3 = jnp.concatenate([edge_index[1], pad]).reshape(NTILES, NCHUNK, 128)
    xn = jnp.transpose(x[0], (1, 0, 2)).reshape(N, T0 * TIN)
    xn = jnp.pad(xn, ((0, NPAD - N), (0, 0)))
    zeros128 = jnp.zeros((NPAD, F1), f32)
    zeros32 = jnp.zeros((NPAD, F2), f32)
    zeros16 = jnp.zeros((NPAD, 16), f32)
    ones16 = jnp.ones((128, 16), f32)
    bnp = jnp.zeros((NPAD, 16), f32)
    bnp = bnp.at[:N, 0].set(p["bn_w"]).at[:N, 1].set(p["bn_b"])

    w1p = _big_conv_w(p["tc1_w1"], T0, TIN, TP, HID)
    w1q = _big_conv_w(p["tc1_w2"], T0, TIN, TP, HID)
    w1r = _big_conv_w(p["tc1_w3"], T0, TIN, TP, HID)
    b1p = _tile_bias(p["tc1_b1"], TP)
    b1q = _tile_bias(p["tc1_b2"], TP)
    b1r = _tile_bias(p["tc1_b3"], TP)
    w0bd = _block_diag(p["cheb1_W"][0], TP)
    w1bd = _block_diag(p["cheb1_W"][1], TP)
    w2bd = _block_diag(p["cheb1_W"][2], TP)
    bch = _tile_bias(p["cheb1_b"], TP)
    w2p = _big_conv_w(p["tc2_w1"], TP, HID, TQ, OUTF, pad_to=F2)
    w2q = _big_conv_w(p["tc2_w2"], TP, HID, TQ, OUTF, pad_to=F2)
    w2r = _big_conv_w(p["tc2_w3"], TP, HID, TQ, OUTF, pad_to=F2)
    b2p = _tile_bias(p["tc2_b1"], TQ, pad_to=F2)
    b2q = _tile_bias(p["tc2_b2"], TQ, pad_to=F2)
    b2r = _tile_bias(p["tc2_b3"], TQ, pad_to=F2)
    wz0 = _block_diag(p["Wxz"][0], TQ, pad_rows=F2)
    wz1 = _block_diag(p["Wxz"][1], TQ, pad_rows=F2)
    bz = _tile_bias(p["bxz"] + p["bhz"], TQ)
    wh0 = _block_diag(p["Wxh"][0], TQ, pad_rows=F2)
    wh1 = _block_diag(p["Wxh"][1], TQ, pad_rows=F2)
    bh = _tile_bias(p["bxh"] + p["bhh"], TQ)
    lwbd = _block_diag(p["lin_W"], TQ, pad_cols=8)
    blin = jnp.broadcast_to(p["lin_b"], (8,)).reshape(1, 8)

    degp = _sc_degree(row3, ones16, zeros16)

    u0, t0 = _tc_call(
        _tc_a_body,
        [xn, degp, w1p, b1p, w1q, b1q, w1r, b1r],
        [_node_spec(T0 * TIN), _part_spec(16),
         _full_spec((T0 * TIN, F1)), _full_spec((1, F1)),
         _full_spec((T0 * TIN, F1)), _full_spec((1, F1)),
         _full_spec((T0 * TIN, F1)), _full_spec((1, F1))],
        [F1, F1],
    )

    g1p = _sc_g128(u0, row3, col3, zeros128)

    u1, o01 = _tc_call(
        _tc_b_body,
        [g1p, t0, degp, w0bd, w1bd],
        [_part_spec(F1), _node_spec(F1), _part_spec(16),
         _full_spec((F1, F1)), _full_spec((F1, F1))],
        [F1, F1],
    )

    g2p = _sc_g128(u1, row3, col3, zeros128)

    h, uh = _tc_call(
        _tc_c_body,
        [g2p, t0, o01, degp, bnp, w2bd, bch, w2p, b2p, w2q, b2q, w2r, b2r],
        [_part_spec(F1), _node_spec(F1), _node_spec(F1), _part_spec(16),
         _node_spec(16), _full_spec((F1, F1)), _full_spec((1, F1)),
         _full_spec((F1, F2)), _full_spec((1, F2)),
         _full_spec((F1, F2)), _full_spec((1, F2)),
         _full_spec((F1, F2)), _full_spec((1, F2))],
        [F2, F2],
    )

    ghp = _sc_g32(uh, row3, col3, zeros32)

    (res,) = _tc_call(
        _tc_d_body,
        [h, ghp, degp, wz0, wz1, bz, wh0, wh1, bh, lwbd, blin],
        [_node_spec(F2), _part_spec(F2), _part_spec(16),
         _full_spec((F2, FZ)), _full_spec((F2, FZ)), _full_spec((1, FZ)),
         _full_spec((F2, FZ)), _full_spec((F2, FZ)), _full_spec((1, FZ)),
         _full_spec((FZ, 8)), _full_spec((1, 8))],
        [8],
    )

    out = res[:N, :TQ]
    return jnp.transpose(out, (1, 0))[None, :, :, None]


# speedup vs baseline: 1.4414x; 1.0384x over previous
import functools

import jax
import jax.numpy as jnp
import numpy as np
from jax import lax
from jax.experimental import pallas as pl
from jax.experimental.pallas import tpu as pltpu
from jax.experimental.pallas import tpu_sc as plsc

N = 10000
NPAD = 10240
E = 160000
EP = 163840
NTILES = 32
NCHUNK = EP // (NTILES * 128)
ZR = NPAD // 16
NB = 2048

T0, TIN, HID, KS = 10, 2, 16, 3
TP = T0 - KS + 1
OUTF = 5
TQ = TP - KS + 1
FILT = 32
F1 = TP * HID
F2 = 32
FZ = TQ * FILT


def _sc_mesh():
    return plsc.VectorSubcoreMesh(core_axis_name="c", subcore_axis_name="s")


@functools.lru_cache(maxsize=None)
def _make_sc_gather_scatter(feat):
    @functools.partial(
        pl.kernel,
        out_type=jax.ShapeDtypeStruct((2, NPAD, feat), jnp.float32),
        mesh=_sc_mesh(),
        compiler_params=pltpu.CompilerParams(
            use_tc_tiling_on_sc=(feat % 128 == 0)),
        scratch_types=[
            pltpu.VMEM((NCHUNK, 128), jnp.int32),
            pltpu.VMEM((NCHUNK, 128), jnp.int32),
            pltpu.VMEM((128, feat), jnp.float32),
            pltpu.VMEM((128, feat), jnp.float32),
            pltpu.VMEM_SHARED((NPAD, feat), jnp.float32),
            pltpu.SemaphoreType.DMA,
            pltpu.SemaphoreType.DMA,
        ],
    )
    def k(u_hbm, row_hbm, col_hbm, zeros_hbm, out_hbm,
          row_v, col_v, buf0, buf1, acc, gs0, gs1):
        cid = lax.axis_index("c")
        sid = lax.axis_index("s")
        w = sid * 2 + cid
        pltpu.sync_copy(zeros_hbm.at[pl.ds(sid * ZR, ZR)], acc.at[pl.ds(sid * ZR, ZR)])
        pltpu.sync_copy(row_hbm.at[w], row_v)
        pltpu.sync_copy(col_hbm.at[w], col_v)
        plsc.subcore_barrier()

        pltpu.async_copy(u_hbm.at[row_v.at[0]], buf0, gs0)

        def body(i, carry):
            j0 = 2 * i
            j1 = 2 * i + 1
            pltpu.make_async_copy(u_hbm.at[row_v.at[j0]], buf0, gs0).wait()
            pltpu.async_copy(u_hbm.at[row_v.at[j1]], buf1, gs1)
            pltpu.sync_copy(buf0, acc.at[col_v.at[j0]], add=True)
            pltpu.make_async_copy(u_hbm.at[row_v.at[j1]], buf1, gs1).wait()

            @pl.when(i < NCHUNK // 2 - 1)
            def _():
                pltpu.async_copy(u_hbm.at[row_v.at[j0 + 2]], buf0, gs0)

            pltpu.sync_copy(buf1, acc.at[col_v.at[j1]], add=True)
            return carry

        lax.fori_loop(0, NCHUNK // 2, body, 0)
        plsc.subcore_barrier()
        pltpu.sync_copy(acc.at[pl.ds(sid * ZR, ZR)],
                        out_hbm.at[cid, pl.ds(sid * ZR, ZR)])

    return k


@functools.lru_cache(maxsize=None)
def _make_sc_degree():
    @functools.partial(
        pl.kernel,
        out_type=jax.ShapeDtypeStruct((2, NPAD, 16), jnp.float32),
        mesh=_sc_mesh(),
        compiler_params=pltpu.CompilerParams(use_tc_tiling_on_sc=False),
        scratch_types=[
            pltpu.VMEM((NCHUNK, 128), jnp.int32),
            pltpu.VMEM((128, 16), jnp.float32),
            pltpu.VMEM_SHARED((NPAD, 16), jnp.float32),
            pltpu.SemaphoreType.DMA,
        ],
    )
    def k(row_hbm, ones_hbm, zeros_hbm, out_hbm, row_v, ones_v, acc, sem):
        cid = lax.axis_index("c")
        sid = lax.axis_index("s")
        w = sid * 2 + cid
        pltpu.sync_copy(zeros_hbm.at[pl.ds(sid * ZR, ZR)], acc.at[pl.ds(sid * ZR, ZR)])
        pltpu.sync_copy(ones_hbm, ones_v)
        pltpu.sync_copy(row_hbm.at[w], row_v)
        plsc.subcore_barrier()

        def body(j, carry):
            pltpu.sync_copy(ones_v, acc.at[row_v.at[j]], add=True)
            return carry

        lax.fori_loop(0, NCHUNK, body, 0)
        plsc.subcore_barrier()
        pltpu.sync_copy(acc.at[pl.ds(sid * ZR, ZR)],
                        out_hbm.at[cid, pl.ds(sid * ZR, ZR)])

    return k


def _sc_degree(row3, ones16, zeros16):
    return _make_sc_degree()(row3, ones16, zeros16)


def _sc_g128(u, row3, col3, zeros):
    return _make_sc_gather_scatter(F1)(u, row3, col3, zeros)


def _sc_g32(u, row3, col3, zeros):
    return _make_sc_gather_scatter(F2)(u, row3, col3, zeros)


def _dis_from(degp):
    d = degp[0, :, 0:1] + degp[1, :, 0:1]
    return jnp.where(d > 0, lax.rsqrt(d), 0.0)


def _tc_a_body(xn, degp, w1p, b1p, w1q, b1q, w1r, b1r, u0_ref, t0_ref):
    x = xn[...]
    dis = _dis_from(degp[...])
    P = jnp.dot(x, w1p[...], preferred_element_type=jnp.float32) + b1p[...]
    Q = jnp.dot(x, w1q[...], preferred_element_type=jnp.float32) + b1q[...]
    Rm = jnp.dot(x, w1r[...], preferred_element_type=jnp.float32) + b1r[...]
    t0 = jax.nn.relu(P * jax.nn.sigmoid(Q) + Rm)
    t0_ref[...] = t0
    u0_ref[...] = dis * t0


def _tc_b_body(g1p, t0, degp, w0bd, w1bd, u1_ref, o01_ref):
    dis = _dis_from(degp[...])
    g1 = g1p[0] + g1p[1]
    tx1 = -dis * g1
    u1_ref[...] = dis * tx1
    o01_ref[...] = (
        jnp.dot(t0[...], w0bd[...], preferred_element_type=jnp.float32)
        + jnp.dot(tx1, w1bd[...], preferred_element_type=jnp.float32)
    )


def _tc_c_body(g2p, t0, o01, degp, bnp, w2bd, bch, w2p, b2p, w2q, b2q, w2r, b2r,
               h_ref, uh_ref):
    dis = _dis_from(degp[...])
    tx2 = -2.0 * dis * (g2p[0] + g2p[1]) - t0[...]
    gfull = jax.nn.relu(
        o01[...] + jnp.dot(tx2, w2bd[...], preferred_element_type=jnp.float32)
        + bch[...]
    )
    P2 = jnp.dot(gfull, w2p[...], preferred_element_type=jnp.float32) + b2p[...]
    Q2 = jnp.dot(gfull, w2q[...], preferred_element_type=jnp.float32) + b2q[...]
    R2 = jnp.dot(gfull, w2r[...], preferred_element_type=jnp.float32) + b2r[...]
    t2 = jax.nn.relu(P2 * jax.nn.sigmoid(Q2) + R2)
    mask = (lax.broadcasted_iota(jnp.int32, (1, F2), 1) < 30).astype(jnp.float32)
    mean = jnp.sum(t2, axis=1, keepdims=True) * (1.0 / 30.0)
    ctr = t2 - mean
    var = jnp.sum(ctr * ctr * mask, axis=1, keepdims=True) * (1.0 / 30.0)
    hh = ctr * lax.rsqrt(var + 1e-5)
    h = jax.nn.relu(hh * bnp[:, 0:1] + bnp[:, 1:2])
    h_ref[...] = h
    uh_ref[...] = dis * h


def _tc_d_body(h, ghp, degp, wz0, wz1, bz, wh0, wh1, bh, lwbd, blin, res_ref):
    dis = _dis_from(degp[...])
    ph = -dis * (ghp[0] + ghp[1])
    hv = h[...]
    Z = jax.nn.sigmoid(
        jnp.dot(hv, wz0[...], preferred_element_type=jnp.float32)
        + jnp.dot(ph, wz1[...], preferred_element_type=jnp.float32) + bz[...]
    )
    Ht = jnp.tanh(
        jnp.dot(hv, wh0[...], preferred_element_type=jnp.float32)
        + jnp.dot(ph, wh1[...], preferred_element_type=jnp.float32) + bh[...]
    )
    H = jax.nn.relu((1.0 - Z) * Ht)
    o = jnp.dot(H, lwbd[...], preferred_element_type=jnp.float32) + blin[...]
    mask6 = lax.broadcasted_iota(jnp.int32, (1, 8), 1) < TQ
    om = jnp.where(mask6, o, -1e30)
    mx = jnp.max(om, axis=1, keepdims=True)
    lse = jnp.log(jnp.sum(jnp.exp(om - mx), axis=1, keepdims=True)) + mx
    res_ref[...] = o - lse


def _node_spec(feat):
    return pl.BlockSpec((NB, feat), lambda i: (i, 0))


def _part_spec(feat):
    return pl.BlockSpec((2, NB, feat), lambda i: (0, i, 0))


def _full_spec(shape):
    nd = len(shape)
    return pl.BlockSpec(shape, lambda i: (0,) * nd)


def _tc_call(body, in_arrays, in_specs, out_feats):
    out_shape = [jax.ShapeDtypeStruct((NPAD, f), jnp.float32) for f in out_feats]
    out_specs = [_node_spec(f) for f in out_feats]
    res = pl.pallas_call(
        body,
        grid=(NPAD // NB,),
        in_specs=in_specs,
        out_specs=out_specs,
        out_shape=out_shape,
    )(*in_arrays)
    return res


def _big_conv_w(w, t_in, c_in, t_out, c_out, pad_to=None):
    rows, cols, oo, ii, kk = [], [], [], [], []
    for ti in range(t_out):
        for k in range(KS):
            for i in range(c_in):
                for o in range(c_out):
                    rows.append((ti + k) * c_in + i)
                    cols.append(ti * c_out + o)
                    oo.append(o)
                    ii.append(i)
                    kk.append(k)
    vals = w[np.array(oo), np.array(ii), 0, np.array(kk)]
    ncol = t_out * c_out if pad_to is None else pad_to
    big = jnp.zeros((t_in * c_in, ncol), jnp.float32)
    return big.at[np.array(rows), np.array(cols)].set(vals)


def _tile_bias(b, t_out, pad_to=None):
    v = jnp.tile(b, t_out)
    if pad_to is not None and pad_to > v.shape[0]:
        v = jnp.pad(v, (0, pad_to - v.shape[0]))
    return v.reshape(1, -1)


def _block_diag(wmat, t, pad_rows=None, pad_cols=None):
    bd = jnp.kron(jnp.eye(t, dtype=jnp.float32), wmat)
    pr = 0 if pad_rows is None else pad_rows - bd.shape[0]
    pc = 0 if pad_cols is None else pad_cols - bd.shape[1]
    if pr or pc:
        bd = jnp.pad(bd, ((0, pr), (0, pc)))
    return bd


def kernel(x, params, edge_index):
    p = params
    f32 = jnp.float32

    pad = jnp.full((EP - E,), N, dtype=jnp.int32)
    row3 = jnp.concatenate([edge_index[0], pad]).reshape(NTILES, NCHUNK, 128)
    col3 = jnp.concatenate([edge_index[1], pad]).reshape(NTILES, NCHUNK, 128)
    xn = jnp.transpose(x[0], (1, 0, 2)).reshape(N, T0 * TIN)
    xn = jnp.pad(xn, ((0, NPAD - N), (0, 0)))
    zeros128 = jnp.zeros((NPAD, F1), f32)
    zeros32 = jnp.zeros((NPAD, F2), f32)
    zeros16 = jnp.zeros((NPAD, 16), f32)
    ones16 = jnp.ones((128, 16), f32)
    bnp = jnp.zeros((NPAD, 16), f32)
    bnp = bnp.at[:N, 0].set(p["bn_w"]).at[:N, 1].set(p["bn_b"])

    w1p = _big_conv_w(p["tc1_w1"], T0, TIN, TP, HID)
    w1q = _big_conv_w(p["tc1_w2"], T0, TIN, TP, HID)
    w1r = _big_conv_w(p["tc1_w3"], T0, TIN, TP, HID)
    b1p = _tile_bias(p["tc1_b1"], TP)
    b1q = _tile_bias(p["tc1_b2"], TP)
    b1r = _tile_bias(p["tc1_b3"], TP)
    w0bd = _block_diag(p["cheb1_W"][0], TP)
    w1bd = _block_diag(p["cheb1_W"][1], TP)
    w2bd = _block_diag(p["cheb1_W"][2], TP)
    bch = _tile_bias(p["cheb1_b"], TP)
    w2p = _big_conv_w(p["tc2_w1"], TP, HID, TQ, OUTF, pad_to=F2)
    w2q = _big_conv_w(p["tc2_w2"], TP, HID, TQ, OUTF, pad_to=F2)
    w2r = _big_conv_w(p["tc2_w3"], TP, HID, TQ, OUTF, pad_to=F2)
    b2p = _tile_bias(p["tc2_b1"], TQ, pad_to=F2)
    b2q = _tile_bias(p["tc2_b2"], TQ, pad_to=F2)
    b2r = _tile_bias(p["tc2_b3"], TQ, pad_to=F2)
    wz0 = _block_diag(p["Wxz"][0], TQ, pad_rows=F2)
    wz1 = _block_diag(p["Wxz"][1], TQ, pad_rows=F2)
    bz = _tile_bias(p["bxz"] + p["bhz"], TQ)
    wh0 = _block_diag(p["Wxh"][0], TQ, pad_rows=F2)
    wh1 = _block_diag(p["Wxh"][1], TQ, pad_rows=F2)
    bh = _tile_bias(p["bxh"] + p["bhh"], TQ)
    lwbd = _block_diag(p["lin_W"], TQ, pad_cols=8)
    blin = jnp.broadcast_to(p["lin_b"], (8,)).reshape(1, 8)

    degp = _sc_degree(row3, ones16, zeros16)

    u0, t0 = _tc_call(
        _tc_a_body,
        [xn, degp, w1p, b1p, w1q, b1q, w1r, b1r],
        [_node_spec(T0 * TIN), _part_spec(16),
         _full_spec((T0 * TIN, F1)), _full_spec((1, F1)),
         _full_spec((T0 * TIN, F1)), _full_spec((1, F1)),
         _full_spec((T0 * TIN, F1)), _full_spec((1, F1))],
        [F1, F1],
    )

    g1p = _sc_g128(u0, row3, col3, zeros128)

    u1, o01 = _tc_call(
        _tc_b_body,
        [g1p, t0, degp, w0bd, w1bd],
        [_part_spec(F1), _node_spec(F1), _part_spec(16),
         _full_spec((F1, F1)), _full_spec((F1, F1))],
        [F1, F1],
    )

    g2p = _sc_g128(u1, row3, col3, zeros128)

    h, uh = _tc_call(
        _tc_c_body,
        [g2p, t0, o01, degp, bnp, w2bd, bch, w2p, b2p, w2q, b2q, w2r, b2r],
        [_part_spec(F1), _node_spec(F1), _node_spec(F1), _part_spec(16),
         _node_spec(16), _full_spec((F1, F1)), _full_spec((1, F1)),
         _full_spec((F1, F2)), _full_spec((1, F2)),
         _full_spec((F1, F2)), _full_spec((1, F2)),
         _full_spec((F1, F2)), _full_spec((1, F2))],
        [F2, F2],
    )

    ghp = _sc_g32(uh, row3, col3, zeros32)

    (res,) = _tc_call(
        _tc_d_body,
        [h, ghp, degp, wz0, wz1, bz, wh0, wh1, bh, lwbd, blin],
        [_node_spec(F2), _part_spec(F2), _part_spec(16),
         _full_spec((F2, FZ)), _full_spec((F2, FZ)), _full_spec((1, FZ)),
         _full_spec((F2, FZ)), _full_spec((F2, FZ)), _full_spec((1, FZ)),
         _full_spec((FZ, 8)), _full_spec((1, 8))],
        [8],
    )

    out = res[:N, :TQ]
    return jnp.transpose(out, (1, 0))[None, :, :, None]


# speedup vs baseline: 1.4494x; 1.0056x over previous
import functools

import jax
import jax.numpy as jnp
import numpy as np
from jax import lax
from jax.experimental import pallas as pl
from jax.experimental.pallas import tpu as pltpu
from jax.experimental.pallas import tpu_sc as plsc

N = 10000
NPAD = 10240
E = 160000
EP = 163840
NTILES = 32
NCHUNK = EP // (NTILES * 128)
ZR = NPAD // 16
NB = 2560

T0, TIN, HID, KS = 10, 2, 16, 3
TP = T0 - KS + 1
OUTF = 5
TQ = TP - KS + 1
FILT = 32
F1 = TP * HID
F2 = 32
FZ = TQ * FILT


def _sc_mesh():
    return plsc.VectorSubcoreMesh(core_axis_name="c", subcore_axis_name="s")


@functools.lru_cache(maxsize=None)
def _make_sc_gather_scatter(feat):
    @functools.partial(
        pl.kernel,
        out_type=jax.ShapeDtypeStruct((2, NPAD, feat), jnp.float32),
        mesh=_sc_mesh(),
        compiler_params=pltpu.CompilerParams(
            use_tc_tiling_on_sc=(feat % 128 == 0)),
        scratch_types=[
            pltpu.VMEM((NCHUNK, 128), jnp.int32),
            pltpu.VMEM((NCHUNK, 128), jnp.int32),
            pltpu.VMEM((128, feat), jnp.float32),
            pltpu.VMEM((128, feat), jnp.float32),
            pltpu.VMEM_SHARED((NPAD, feat), jnp.float32),
            pltpu.SemaphoreType.DMA,
            pltpu.SemaphoreType.DMA,
        ],
    )
    def k(u_hbm, row_hbm, col_hbm, zeros_hbm, out_hbm,
          row_v, col_v, buf0, buf1, acc, gs0, gs1):
        cid = lax.axis_index("c")
        sid = lax.axis_index("s")
        w = sid * 2 + cid
        pltpu.sync_copy(zeros_hbm.at[pl.ds(sid * ZR, ZR)], acc.at[pl.ds(sid * ZR, ZR)])
        pltpu.sync_copy(row_hbm.at[w], row_v)
        pltpu.sync_copy(col_hbm.at[w], col_v)
        plsc.subcore_barrier()

        pltpu.async_copy(u_hbm.at[row_v.at[0]], buf0, gs0)

        def body(i, carry):
            j0 = 2 * i
            j1 = 2 * i + 1
            pltpu.make_async_copy(u_hbm.at[row_v.at[j0]], buf0, gs0).wait()
            pltpu.async_copy(u_hbm.at[row_v.at[j1]], buf1, gs1)
            pltpu.sync_copy(buf0, acc.at[col_v.at[j0]], add=True)
            pltpu.make_async_copy(u_hbm.at[row_v.at[j1]], buf1, gs1).wait()

            @pl.when(i < NCHUNK // 2 - 1)
            def _():
                pltpu.async_copy(u_hbm.at[row_v.at[j0 + 2]], buf0, gs0)

            pltpu.sync_copy(buf1, acc.at[col_v.at[j1]], add=True)
            return carry

        lax.fori_loop(0, NCHUNK // 2, body, 0)
        plsc.subcore_barrier()
        pltpu.sync_copy(acc.at[pl.ds(sid * ZR, ZR)],
                        out_hbm.at[cid, pl.ds(sid * ZR, ZR)])

    return k


@functools.lru_cache(maxsize=None)
def _make_sc_degree():
    @functools.partial(
        pl.kernel,
        out_type=jax.ShapeDtypeStruct((2, NPAD, 16), jnp.float32),
        mesh=_sc_mesh(),
        compiler_params=pltpu.CompilerParams(use_tc_tiling_on_sc=False),
        scratch_types=[
            pltpu.VMEM((NCHUNK, 128), jnp.int32),
            pltpu.VMEM((128, 16), jnp.float32),
            pltpu.VMEM_SHARED((NPAD, 16), jnp.float32),
            pltpu.SemaphoreType.DMA,
        ],
    )
    def k(row_hbm, ones_hbm, zeros_hbm, out_hbm, row_v, ones_v, acc, sem):
        cid = lax.axis_index("c")
        sid = lax.axis_index("s")
        w = sid * 2 + cid
        pltpu.sync_copy(zeros_hbm.at[pl.ds(sid * ZR, ZR)], acc.at[pl.ds(sid * ZR, ZR)])
        pltpu.sync_copy(ones_hbm, ones_v)
        pltpu.sync_copy(row_hbm.at[w], row_v)
        plsc.subcore_barrier()

        def body(j, carry):
            pltpu.sync_copy(ones_v, acc.at[row_v.at[j]], add=True)
            return carry

        lax.fori_loop(0, NCHUNK, body, 0)
        plsc.subcore_barrier()
        pltpu.sync_copy(acc.at[pl.ds(sid * ZR, ZR)],
                        out_hbm.at[cid, pl.ds(sid * ZR, ZR)])

    return k


def _sc_degree(row3, ones16, zeros16):
    return _make_sc_degree()(row3, ones16, zeros16)


def _sc_g128(u, row3, col3, zeros):
    return _make_sc_gather_scatter(F1)(u, row3, col3, zeros)


def _sc_g32(u, row3, col3, zeros):
    return _make_sc_gather_scatter(F2)(u, row3, col3, zeros)


def _dis_from(degp):
    d = degp[0, :, 0:1] + degp[1, :, 0:1]
    return jnp.where(d > 0, lax.rsqrt(d), 0.0)


def _tc_a_body(xn, degp, w1p, b1p, w1q, b1q, w1r, b1r, u0_ref, t0_ref):
    x = xn[...]
    dis = _dis_from(degp[...])
    P = jnp.dot(x, w1p[...], preferred_element_type=jnp.float32) + b1p[...]
    Q = jnp.dot(x, w1q[...], preferred_element_type=jnp.float32) + b1q[...]
    Rm = jnp.dot(x, w1r[...], preferred_element_type=jnp.float32) + b1r[...]
    t0 = jax.nn.relu(P * jax.nn.sigmoid(Q) + Rm)
    t0_ref[...] = t0
    u0_ref[...] = dis * t0


def _tc_b_body(g1p, t0, degp, w0bd, w1bd, u1_ref, o01_ref):
    dis = _dis_from(degp[...])
    g1 = g1p[0] + g1p[1]
    tx1 = -dis * g1
    u1_ref[...] = dis * tx1
    o01_ref[...] = (
        jnp.dot(t0[...], w0bd[...], preferred_element_type=jnp.float32)
        + jnp.dot(tx1, w1bd[...], preferred_element_type=jnp.float32)
    )


def _tc_c_body(g2p, t0, o01, degp, bnp, w2bd, bch, w2p, b2p, w2q, b2q, w2r, b2r,
               h_ref, uh_ref):
    dis = _dis_from(degp[...])
    tx2 = -2.0 * dis * (g2p[0] + g2p[1]) - t0[...]
    gfull = jax.nn.relu(
        o01[...] + jnp.dot(tx2, w2bd[...], preferred_element_type=jnp.float32)
        + bch[...]
    )
    P2 = jnp.dot(gfull, w2p[...], preferred_element_type=jnp.float32) + b2p[...]
    Q2 = jnp.dot(gfull, w2q[...], preferred_element_type=jnp.float32) + b2q[...]
    R2 = jnp.dot(gfull, w2r[...], preferred_element_type=jnp.float32) + b2r[...]
    t2 = jax.nn.relu(P2 * jax.nn.sigmoid(Q2) + R2)
    mask = (lax.broadcasted_iota(jnp.int32, (1, F2), 1) < 30).astype(jnp.float32)
    mean = jnp.sum(t2, axis=1, keepdims=True) * (1.0 / 30.0)
    ctr = t2 - mean
    var = jnp.sum(ctr * ctr * mask, axis=1, keepdims=True) * (1.0 / 30.0)
    hh = ctr * lax.rsqrt(var + 1e-5)
    h = jax.nn.relu(hh * bnp[:, 0:1] + bnp[:, 1:2])
    h_ref[...] = h
    uh_ref[...] = dis * h


def _tc_d_body(h, ghp, degp, wz0, wz1, bz, wh0, wh1, bh, lwbd, blin, res_ref):
    dis = _dis_from(degp[...])
    ph = -dis * (ghp[0] + ghp[1])
    hv = h[...]
    Z = jax.nn.sigmoid(
        jnp.dot(hv, wz0[...], preferred_element_type=jnp.float32)
        + jnp.dot(ph, wz1[...], preferred_element_type=jnp.float32) + bz[...]
    )
    Ht = jnp.tanh(
        jnp.dot(hv, wh0[...], preferred_element_type=jnp.float32)
        + jnp.dot(ph, wh1[...], preferred_element_type=jnp.float32) + bh[...]
    )
    H = jax.nn.relu((1.0 - Z) * Ht)
    o = jnp.dot(H, lwbd[...], preferred_element_type=jnp.float32) + blin[...]
    mask6 = lax.broadcasted_iota(jnp.int32, (1, 8), 1) < TQ
    om = jnp.where(mask6, o, -1e30)
    mx = jnp.max(om, axis=1, keepdims=True)
    lse = jnp.log(jnp.sum(jnp.exp(om - mx), axis=1, keepdims=True)) + mx
    res_ref[...] = o - lse


def _node_spec(feat):
    return pl.BlockSpec((NB, feat), lambda i: (i, 0))


def _part_spec(feat):
    return pl.BlockSpec((2, NB, feat), lambda i: (0, i, 0))


def _full_spec(shape):
    nd = len(shape)
    return pl.BlockSpec(shape, lambda i: (0,) * nd)


def _tc_call(body, in_arrays, in_specs, out_feats):
    out_shape = [jax.ShapeDtypeStruct((NPAD, f), jnp.float32) for f in out_feats]
    out_specs = [_node_spec(f) for f in out_feats]
    res = pl.pallas_call(
        body,
        grid=(NPAD // NB,),
        in_specs=in_specs,
        out_specs=out_specs,
        out_shape=out_shape,
    )(*in_arrays)
    return res


def _big_conv_w(w, t_in, c_in, t_out, c_out, pad_to=None):
    rows, cols, oo, ii, kk = [], [], [], [], []
    for ti in range(t_out):
        for k in range(KS):
            for i in range(c_in):
                for o in range(c_out):
                    rows.append((ti + k) * c_in + i)
                    cols.append(ti * c_out + o)
                    oo.append(o)
                    ii.append(i)
                    kk.append(k)
    vals = w[np.array(oo), np.array(ii), 0, np.array(kk)]
    ncol = t_out * c_out if pad_to is None else pad_to
    big = jnp.zeros((t_in * c_in, ncol), jnp.float32)
    return big.at[np.array(rows), np.array(cols)].set(vals)


def _tile_bias(b, t_out, pad_to=None):
    v = jnp.tile(b, t_out)
    if pad_to is not None and pad_to > v.shape[0]:
        v = jnp.pad(v, (0, pad_to - v.shape[0]))
    return v.reshape(1, -1)


def _block_diag(wmat, t, pad_rows=None, pad_cols=None):
    bd = jnp.kron(jnp.eye(t, dtype=jnp.float32), wmat)
    pr = 0 if pad_rows is None else pad_rows - bd.shape[0]
    pc = 0 if pad_cols is None else pad_cols - bd.shape[1]
    if pr or pc:
        bd = jnp.pad(bd, ((0, pr), (0, pc)))
    return bd


def kernel(x, params, edge_index):
    p = params
    f32 = jnp.float32

    pad = jnp.full((EP - E,), N, dtype=jnp.int32)
    row3 = jnp.concatenate([edge_index[0], pad]).reshape(NTILES, NCHUNK, 128)
    col3 = jnp.concatenate([edge_index[1], pad]).reshape(NTILES, NCHUNK, 128)
    xn = jnp.transpose(x[0], (1, 0, 2)).reshape(N, T0 * TIN)
    xn = jnp.pad(xn, ((0, NPAD - N), (0, 0)))
    zeros128 = jnp.zeros((NPAD, F1), f32)
    zeros32 = jnp.zeros((NPAD, F2), f32)
    zeros16 = jnp.zeros((NPAD, 16), f32)
    ones16 = jnp.ones((128, 16), f32)
    bnp = jnp.zeros((NPAD, 16), f32)
    bnp = bnp.at[:N, 0].set(p["bn_w"]).at[:N, 1].set(p["bn_b"])

    w1p = _big_conv_w(p["tc1_w1"], T0, TIN, TP, HID)
    w1q = _big_conv_w(p["tc1_w2"], T0, TIN, TP, HID)
    w1r = _big_conv_w(p["tc1_w3"], T0, TIN, TP, HID)
    b1p = _tile_bias(p["tc1_b1"], TP)
    b1q = _tile_bias(p["tc1_b2"], TP)
    b1r = _tile_bias(p["tc1_b3"], TP)
    w0bd = _block_diag(p["cheb1_W"][0], TP)
    w1bd = _block_diag(p["cheb1_W"][1], TP)
    w2bd = _block_diag(p["cheb1_W"][2], TP)
    bch = _tile_bias(p["cheb1_b"], TP)
    w2p = _big_conv_w(p["tc2_w1"], TP, HID, TQ, OUTF, pad_to=F2)
    w2q = _big_conv_w(p["tc2_w2"], TP, HID, TQ, OUTF, pad_to=F2)
    w2r = _big_conv_w(p["tc2_w3"], TP, HID, TQ, OUTF, pad_to=F2)
    b2p = _tile_bias(p["tc2_b1"], TQ, pad_to=F2)
    b2q = _tile_bias(p["tc2_b2"], TQ, pad_to=F2)
    b2r = _tile_bias(p["tc2_b3"], TQ, pad_to=F2)
    wz0 = _block_diag(p["Wxz"][0], TQ, pad_rows=F2)
    wz1 = _block_diag(p["Wxz"][1], TQ, pad_rows=F2)
    bz = _tile_bias(p["bxz"] + p["bhz"], TQ)
    wh0 = _block_diag(p["Wxh"][0], TQ, pad_rows=F2)
    wh1 = _block_diag(p["Wxh"][1], TQ, pad_rows=F2)
    bh = _tile_bias(p["bxh"] + p["bhh"], TQ)
    lwbd = _block_diag(p["lin_W"], TQ, pad_cols=8)
    blin = jnp.broadcast_to(p["lin_b"], (8,)).reshape(1, 8)

    degp = _sc_degree(row3, ones16, zeros16)

    u0, t0 = _tc_call(
        _tc_a_body,
        [xn, degp, w1p, b1p, w1q, b1q, w1r, b1r],
        [_node_spec(T0 * TIN), _part_spec(16),
         _full_spec((T0 * TIN, F1)), _full_spec((1, F1)),
         _full_spec((T0 * TIN, F1)), _full_spec((1, F1)),
         _full_spec((T0 * TIN, F1)), _full_spec((1, F1))],
        [F1, F1],
    )

    g1p = _sc_g128(u0, row3, col3, zeros128)

    u1, o01 = _tc_call(
        _tc_b_body,
        [g1p, t0, degp, w0bd, w1bd],
        [_part_spec(F1), _node_spec(F1), _part_spec(16),
         _full_spec((F1, F1)), _full_spec((F1, F1))],
        [F1, F1],
    )

    g2p = _sc_g128(u1, row3, col3, zeros128)

    h, uh = _tc_call(
        _tc_c_body,
        [g2p, t0, o01, degp, bnp, w2bd, bch, w2p, b2p, w2q, b2q, w2r, b2r],
        [_part_spec(F1), _node_spec(F1), _node_spec(F1), _part_spec(16),
         _node_spec(16), _full_spec((F1, F1)), _full_spec((1, F1)),
         _full_spec((F1, F2)), _full_spec((1, F2)),
         _full_spec((F1, F2)), _full_spec((1, F2)),
         _full_spec((F1, F2)), _full_spec((1, F2))],
        [F2, F2],
    )

    ghp = _sc_g32(uh, row3, col3, zeros32)

    (res,) = _tc_call(
        _tc_d_body,
        [h, ghp, degp, wz0, wz1, bz, wh0, wh1, bh, lwbd, blin],
        [_node_spec(F2), _part_spec(F2), _part_spec(16),
         _full_spec((F2, FZ)), _full_spec((F2, FZ)), _full_spec((1, FZ)),
         _full_spec((F2, FZ)), _full_spec((F2, FZ)), _full_spec((1, FZ)),
         _full_spec((FZ, 8)), _full_spec((1, 8))],
        [8],
    )

    out = res[:N, :TQ]
    return jnp.transpose(out, (1, 0))[None, :, :, None]
